# baseline (speedup 1.0000x reference)
import functools

import jax
import jax.numpy as jnp
from jax import lax
from jax.experimental import pallas as pl
from jax.experimental.pallas import tpu as pltpu
from jax.experimental.pallas import tpu_sc as plsc

SCALE = 8.0


@functools.cache
def _build(B, V2, D):
    info = plsc.get_sparse_core_info()
    NC, NS, L = info.num_cores, info.num_subcores, info.num_lanes
    NW = NC * NS
    b_per_w = B // NW
    C = 320
    n_chunks = b_per_w // C
    W = 2 * D
    nj = D // L
    assert b_per_w % C == 0 and C % 16 == 0

    mesh = plsc.VectorSubcoreMesh(core_axis_name="c", subcore_axis_name="s")

    @functools.partial(
        pl.kernel,
        mesh=mesh,
        out_type=jax.ShapeDtypeStruct((B // 2, W), jnp.float32),
        scratch_types=[
            pltpu.VMEM((b_per_w,), jnp.int32),
            pltpu.VMEM((C,), jnp.int32),
            pltpu.VMEM((C,), jnp.int32),
            pltpu.VMEM((C, W), jnp.float32),
            pltpu.VMEM((C, W), jnp.float32),
            pltpu.SemaphoreType.DMA,
            pltpu.SemaphoreType.DMA,
            pltpu.SemaphoreType.DMA,
            pltpu.SemaphoreType.DMA,
        ],
        compiler_params=pltpu.CompilerParams(use_tc_tiling_on_sc=True),
    )
    def gather_scale(idx_hbm, table_hbm, out_hbm,
                     idx_v, ih0, ih1, rows0, rows1, g0, g1, o0, o1):
        wid = lax.axis_index("s") * NC + lax.axis_index("c")
        base = pl.multiple_of(wid * b_per_w, b_per_w)
        bufs = (rows0, rows1)
        ihs = (ih0, ih1)
        gsems = (g0, g1)
        osems = (o0, o1)

        pltpu.sync_copy(idx_hbm.at[pl.ds(base, b_per_w)], idx_v)

        def start_gather(ci, b):
            ih = ihs[b]

            def shr(jb, c):
                sl = pl.ds(jb * L, L)
                ih[sl] = lax.shift_right_logical(
                    idx_v[pl.ds(ci * C + jb * L, L)], 1)
                return c

            lax.fori_loop(0, C // L, shr, 0)
            pltpu.async_copy(table_hbm.at[ih], bufs[b], gsems[b])

        def gather_wait(b):
            pltpu.make_async_copy(
                table_hbm.at[ihs[b]], bufs[b], gsems[b]).wait()

        def select_scale(ci, buf):
            def body(g, c):
                r0 = g * L
                sv = lax.shift_left(idx_v[pl.ds(ci * C + r0, L)] & 1, 6)
                for u in range(L):
                    src = sv[u]
                    for j in range(nj):
                        sl = pl.ds(src + j * L, L)
                        dst = pl.ds((u % 2) * D + j * L, L)
                        buf[g * (L // 2) + u // 2, dst] = buf[r0 + u, sl] * SCALE
                return c

            lax.fori_loop(0, C // L, body, 0)

        def start_out(ci, b):
            obase = pl.multiple_of((base + ci * C) // 2, C // 2)
            pltpu.async_copy(
                bufs[b].at[pl.ds(0, C // 2)],
                out_hbm.at[pl.ds(obase, C // 2)], osems[b])

        def out_wait(b):
            pltpu.make_async_copy(
                bufs[b].at[pl.ds(0, C // 2)],
                out_hbm.at[pl.ds(0, C // 2)], osems[b]).wait()

        n_pairs = n_chunks // 2
        start_gather(0, 0)
        start_gather(1, 1)

        def pair_body(k, c):
            c0 = 2 * k
            gather_wait(0)
            select_scale(c0, rows0)
            start_out(c0, 0)
            gather_wait(1)
            select_scale(c0 + 1, rows1)
            out_wait(0)
            start_gather(c0 + 2, 0)
            start_out(c0 + 1, 1)
            out_wait(1)
            start_gather(c0 + 3, 1)
            return c

        lax.fori_loop(0, n_pairs - 1, pair_body, 0)
        cl = n_chunks - 2
        gather_wait(0)
        select_scale(cl, rows0)
        start_out(cl, 0)
        gather_wait(1)
        select_scale(cl + 1, rows1)
        start_out(cl + 1, 1)
        out_wait(0)
        out_wait(1)

    return gather_scale


def kernel(x, lut):
    Bt, S = x.shape
    B = Bt * S
    V, D = lut.shape
    xflat = x.reshape(B).astype(jnp.int32)
    lut2 = lut.reshape(V // 2, 2 * D)
    out2 = _build(B, V // 2, D)(xflat, lut2)
    return out2.reshape(Bt, S, D)


# speedup vs baseline: 1.0823x; 1.0823x over previous
import functools

import jax
import jax.numpy as jnp
from jax import lax
from jax.experimental import pallas as pl
from jax.experimental.pallas import tpu as pltpu
from jax.experimental.pallas import tpu_sc as plsc

SCALE = 8.0


@functools.cache
def _build(B, V, D):
    info = plsc.get_sparse_core_info()
    NC, NS, L = info.num_cores, info.num_subcores, info.num_lanes
    NW = NC * NS
    b_per_w = B // NW
    C = 320
    n_chunks = b_per_w // C
    W = 2 * D
    nj = D // L
    assert b_per_w % C == 0 and C % L == 0 and n_chunks % 2 == 0

    mesh = plsc.VectorSubcoreMesh(core_axis_name="c", subcore_axis_name="s")

    @functools.partial(
        pl.kernel,
        mesh=mesh,
        out_type=jax.ShapeDtypeStruct((B // 2, W), jnp.float32),
        scratch_types=[
            pltpu.VMEM((b_per_w,), jnp.int32),
            pltpu.VMEM((C, W), jnp.float32),
            pltpu.VMEM((C, W), jnp.float32),
            pltpu.SemaphoreType.DMA,
            pltpu.SemaphoreType.DMA,
            pltpu.SemaphoreType.DMA,
            pltpu.SemaphoreType.DMA,
        ],
        compiler_params=pltpu.CompilerParams(use_tc_tiling_on_sc=True),
    )
    def gather_scale(idx_hbm, table_hbm, out_hbm,
                     idx_v, rows0, rows1, g0, g1, o0, o1):
        wid = lax.axis_index("s") * NC + lax.axis_index("c")
        base = pl.multiple_of(wid * b_per_w, b_per_w)
        bufs = (rows0, rows1)
        gsems = (g0, g1)
        osems = (o0, o1)

        pltpu.sync_copy(idx_hbm.at[pl.ds(base, b_per_w)], idx_v)

        def start_gather(ci, b):
            pltpu.async_copy(
                table_hbm.at[idx_v.at[pl.ds(ci * C, C)]], bufs[b], gsems[b])

        def gather_wait(ci, b):
            pltpu.make_async_copy(
                table_hbm.at[idx_v.at[pl.ds(ci * C, C)]], bufs[b],
                gsems[b]).wait()

        def select_scale(buf):
            def body(g, c):
                r0 = g * L
                for u in range(L):
                    for j in range(nj):
                        src = pl.ds(j * L, L)
                        dst = pl.ds((u % 2) * D + j * L, L)
                        buf[g * (L // 2) + u // 2, dst] = buf[r0 + u, src] * SCALE
                return c

            lax.fori_loop(0, C // L, body, 0)

        def start_out(ci, b):
            obase = pl.multiple_of((base + ci * C) // 2, C // 2)
            pltpu.async_copy(
                bufs[b].at[pl.ds(0, C // 2)],
                out_hbm.at[pl.ds(obase, C // 2)], osems[b])

        def out_wait(b):
            pltpu.make_async_copy(
                bufs[b].at[pl.ds(0, C // 2)],
                out_hbm.at[pl.ds(0, C // 2)], osems[b]).wait()

        n_pairs = n_chunks // 2
        start_gather(0, 0)
        start_gather(1, 1)

        def pair_body(k, c):
            c0 = 2 * k
            gather_wait(c0, 0)
            select_scale(rows0)
            start_out(c0, 0)
            gather_wait(c0 + 1, 1)
            select_scale(rows1)
            out_wait(0)
            start_gather(c0 + 2, 0)
            start_out(c0 + 1, 1)
            out_wait(1)
            start_gather(c0 + 3, 1)
            return c

        lax.fori_loop(0, n_pairs - 1, pair_body, 0)
        cl = n_chunks - 2
        gather_wait(cl, 0)
        select_scale(rows0)
        start_out(cl, 0)
        gather_wait(cl + 1, 1)
        select_scale(rows1)
        start_out(cl + 1, 1)
        out_wait(0)
        out_wait(1)

    return gather_scale


def kernel(x, lut):
    Bt, S = x.shape
    B = Bt * S
    V, D = lut.shape
    xflat = x.reshape(B).astype(jnp.int32)
    lutw = jnp.pad(lut, ((0, 0), (0, D)))
    out2 = _build(B, V, D)(xflat, lutw)
    return out2.reshape(Bt, S, D)


# speedup vs baseline: 1.3713x; 1.2671x over previous
import functools

import jax
import jax.numpy as jnp
from jax import lax
from jax.experimental import pallas as pl
from jax.experimental.pallas import tpu as pltpu
from jax.experimental.pallas import tpu_sc as plsc

SCALE = 8.0


@functools.cache
def _build(B, V, D):
    info = plsc.get_sparse_core_info()
    NC, NS, L = info.num_cores, info.num_subcores, info.num_lanes
    NW = NC * NS
    b_per_w = B // NW
    C = 640
    n_chunks = b_per_w // C
    G = D // L
    M = 2 * D // L
    assert b_per_w % C == 0 and C % L == 0 and n_chunks % 2 == 0

    mesh = plsc.VectorSubcoreMesh(core_axis_name="c", subcore_axis_name="s")

    @functools.partial(
        pl.kernel,
        mesh=mesh,
        out_type=jax.ShapeDtypeStruct((B * G, L), jnp.float32),
        scratch_types=[
            pltpu.VMEM((b_per_w,), jnp.int32),
            pltpu.VMEM((C * G,), jnp.int32),
            pltpu.VMEM((C * G,), jnp.int32),
            pltpu.VMEM((C * G, L), jnp.float32),
            pltpu.VMEM((C * G, L), jnp.float32),
            pltpu.SemaphoreType.DMA,
            pltpu.SemaphoreType.DMA,
            pltpu.SemaphoreType.DMA,
            pltpu.SemaphoreType.DMA,
        ],
        compiler_params=pltpu.CompilerParams(
            use_tc_tiling_on_sc=False, needs_layout_passes=False),
    )
    def gather_scale(idx_hbm, table_hbm, out_hbm,
                     idx_v, i40, i41, rows0, rows1, g0, g1, o0, o1):
        wid = lax.axis_index("s") * NC + lax.axis_index("c")
        base = pl.multiple_of(wid * b_per_w, b_per_w)
        bufs = (rows0, rows1)
        i4s = (i40, i41)
        gsems = (g0, g1)
        osems = (o0, o1)

        pltpu.sync_copy(idx_hbm.at[pl.ds(base, b_per_w)], idx_v)

        iota = lax.iota(jnp.int32, L)
        rks = [lax.shift_right_logical(iota + L * k, 2) for k in range(G)]
        tks = [(iota + L * k) & 3 for k in range(G)]

        def start_gather(ci, b):
            i4 = i4s[b]

            def grp(g, c):
                gb = ci * C + g * L
                for k in range(G):
                    vals = plsc.load_gather(idx_v, [gb + rks[k]])
                    i4[pl.ds(G * L * g + L * k, L)] = vals * M + tks[k]
                return c

            lax.fori_loop(0, C // L, grp, 0)
            pltpu.async_copy(table_hbm.at[i4], bufs[b], gsems[b])

        def gather_wait(b):
            pltpu.make_async_copy(
                table_hbm.at[i4s[b]], bufs[b], gsems[b]).wait()

        def scale(buf):
            def body(q, c):
                r0 = q * L
                for u in range(L):
                    buf[r0 + u, :] = buf[r0 + u, :] * SCALE
                return c

            lax.fori_loop(0, C * G // L, body, 0)

        def start_out(ci, b):
            obase = pl.multiple_of((base + ci * C) * G, C * G)
            pltpu.async_copy(
                bufs[b], out_hbm.at[pl.ds(obase, C * G)], osems[b])

        def out_wait(b):
            pltpu.make_async_copy(
                bufs[b], out_hbm.at[pl.ds(0, C * G)], osems[b]).wait()

        n_pairs = n_chunks // 2
        start_gather(0, 0)
        start_gather(1, 1)

        def pair_body(k, c):
            c0 = 2 * k
            gather_wait(0)
            scale(rows0)
            start_out(c0, 0)
            gather_wait(1)
            scale(rows1)
            out_wait(0)
            start_gather(c0 + 2, 0)
            start_out(c0 + 1, 1)
            out_wait(1)
            start_gather(c0 + 3, 1)
            return c

        lax.fori_loop(0, n_pairs - 1, pair_body, 0)
        cl = n_chunks - 2
        gather_wait(0)
        scale(rows0)
        start_out(cl, 0)
        gather_wait(1)
        scale(rows1)
        start_out(cl + 1, 1)
        out_wait(0)
        out_wait(1)

    return gather_scale


@functools.cache
def _build_tc_transpose(D, V, BK):
    grid = (V + BK - 1) // BK

    def body(in_ref, out_ref):
        t = jnp.transpose(in_ref[...])
        out_ref[:, 0:D] = t
        out_ref[:, D:2 * D] = jnp.zeros((BK, D), jnp.float32)

    return pl.pallas_call(
        body,
        grid=(grid,),
        in_specs=[pl.BlockSpec((D, BK), lambda g: (0, g))],
        out_specs=pl.BlockSpec((BK, 2 * D), lambda g: (g, 0)),
        out_shape=jax.ShapeDtypeStruct((V, 2 * D), jnp.float32),
    )


def kernel(x, lut):
    Bt, S = x.shape
    B = Bt * S
    V, D = lut.shape
    xflat = x.reshape(B).astype(jnp.int32)
    tabw = _build_tc_transpose(D, V, 2048)(lut.T)
    tab16 = tabw.reshape(V * (2 * D // 16), 16)
    out4 = _build(B, V, D)(xflat, tab16)
    return out4.reshape(Bt, S, D)


# speedup vs baseline: 1.9755x; 1.4406x over previous
import functools

import jax
import jax.numpy as jnp
from jax import lax
from jax.experimental import pallas as pl
from jax.experimental.pallas import tpu as pltpu
from jax.experimental.pallas import tpu_sc as plsc

SCALE = 8.0
BK = 8192
HB = BK // 2


@functools.cache
def _build_tc_transpose(D, V):
    grid = (V + BK - 1) // BK

    def body(in_ref, out_ref):
        out_ref[:, 0:D] = jnp.transpose(in_ref[:, 0:HB])
        out_ref[:, D:2 * D] = jnp.transpose(in_ref[:, HB:BK])

    return pl.pallas_call(
        body,
        grid=(grid,),
        in_specs=[pl.BlockSpec((D, BK), lambda g: (0, g))],
        out_specs=pl.BlockSpec((HB, 2 * D), lambda g: (g, 0)),
        out_shape=jax.ShapeDtypeStruct((grid * HB, 2 * D), jnp.float32),
    )


@functools.cache
def _build(B, V, D):
    info = plsc.get_sparse_core_info()
    NC, NS, L = info.num_cores, info.num_subcores, info.num_lanes
    NW = NC * NS
    b_per_w = B // NW
    C = 640
    n_chunks = b_per_w // C
    G = D // L
    M = 2 * D // L
    assert b_per_w % C == 0 and C % L == 0 and n_chunks % 2 == 0

    mesh = plsc.VectorSubcoreMesh(core_axis_name="c", subcore_axis_name="s")

    @functools.partial(
        pl.kernel,
        mesh=mesh,
        out_type=jax.ShapeDtypeStruct((B * G, L), jnp.float32),
        scratch_types=[
            pltpu.VMEM((b_per_w,), jnp.int32),
            pltpu.VMEM((C * G,), jnp.int32),
            pltpu.VMEM((C * G,), jnp.int32),
            pltpu.VMEM((C * G, L), jnp.float32),
            pltpu.VMEM((C * G, L), jnp.float32),
            pltpu.SemaphoreType.DMA,
            pltpu.SemaphoreType.DMA,
            pltpu.SemaphoreType.DMA,
            pltpu.SemaphoreType.DMA,
        ],
        compiler_params=pltpu.CompilerParams(
            use_tc_tiling_on_sc=False, needs_layout_passes=False),
    )
    def gather_scale(idx_hbm, table_hbm, out_hbm,
                     idx_v, i40, i41, rows0, rows1, g0, g1, o0, o1):
        wid = lax.axis_index("s") * NC + lax.axis_index("c")
        base = pl.multiple_of(wid * b_per_w, b_per_w)
        bufs = (rows0, rows1)
        i4s = (i40, i41)
        gsems = (g0, g1)
        osems = (o0, o1)

        pltpu.sync_copy(idx_hbm.at[pl.ds(base, b_per_w)], idx_v)

        iota = lax.iota(jnp.int32, L)
        rks = [lax.shift_right_logical(iota + L * k, 2) for k in range(G)]
        tks = [(iota + L * k) & 3 for k in range(G)]

        def start_gather(ci, b):
            i4 = i4s[b]

            def grp(g, c):
                gb = ci * C + g * L
                for k in range(G):
                    v = plsc.load_gather(idx_v, [gb + rks[k]])
                    row = lax.shift_left(lax.shift_right_logical(v, 13), 12) \
                        + (v & (HB - 1))
                    h = lax.shift_right_logical(v, 12) & 1
                    i4[pl.ds(G * L * g + L * k, L)] = (
                        lax.shift_left(row, 3) + lax.shift_left(h, 2) + tks[k])
                return c

            lax.fori_loop(0, C // L, grp, 0)
            pltpu.async_copy(table_hbm.at[i4], bufs[b], gsems[b])

        def gather_wait(b):
            pltpu.make_async_copy(
                table_hbm.at[i4s[b]], bufs[b], gsems[b]).wait()

        def scale(buf):
            def body(q, c):
                r0 = q * L
                for u in range(L):
                    buf[r0 + u, :] = buf[r0 + u, :] * SCALE
                return c

            lax.fori_loop(0, C * G // L, body, 0)

        def start_out(ci, b):
            obase = pl.multiple_of((base + ci * C) * G, C * G)
            pltpu.async_copy(
                bufs[b], out_hbm.at[pl.ds(obase, C * G)], osems[b])

        def out_wait(b):
            pltpu.make_async_copy(
                bufs[b], out_hbm.at[pl.ds(0, C * G)], osems[b]).wait()

        n_pairs = n_chunks // 2
        start_gather(0, 0)
        start_gather(1, 1)

        def pair_body(k, c):
            c0 = 2 * k
            gather_wait(0)
            scale(rows0)
            start_out(c0, 0)
            gather_wait(1)
            scale(rows1)
            out_wait(0)
            start_gather(c0 + 2, 0)
            start_out(c0 + 1, 1)
            out_wait(1)
            start_gather(c0 + 3, 1)
            return c

        lax.fori_loop(0, n_pairs - 1, pair_body, 0)
        cl = n_chunks - 2
        gather_wait(0)
        scale(rows0)
        start_out(cl, 0)
        gather_wait(1)
        scale(rows1)
        start_out(cl + 1, 1)
        out_wait(0)
        out_wait(1)

    return gather_scale


def kernel(x, lut):
    Bt, S = x.shape
    B = Bt * S
    V, D = lut.shape
    xflat = x.reshape(B).astype(jnp.int32)
    tabw = _build_tc_transpose(D, V)(lut.T)
    tab16 = tabw.reshape(tabw.shape[0] * (2 * D // 16), 16)
    out4 = _build(B, V, D)(xflat, tab16)
    return out4.reshape(Bt, S, D)


# speedup vs baseline: 2.1225x; 1.0744x over previous
import functools

import jax
import jax.numpy as jnp
from jax import lax
from jax.experimental import pallas as pl
from jax.experimental.pallas import tpu as pltpu
from jax.experimental.pallas import tpu_sc as plsc

SCALE = 8.0
BK = 16384
HB = BK // 2


@functools.cache
def _build_tc_transpose(D, V):
    grid = (V + BK - 1) // BK

    def body(in_ref, out_ref):
        out_ref[:, 0:D] = jnp.transpose(in_ref[:, 0:HB])
        out_ref[:, D:2 * D] = jnp.transpose(in_ref[:, HB:BK])

    return pl.pallas_call(
        body,
        grid=(grid,),
        in_specs=[pl.BlockSpec((D, BK), lambda g: (0, g))],
        out_specs=pl.BlockSpec((HB, 2 * D), lambda g: (g, 0)),
        out_shape=jax.ShapeDtypeStruct((grid * HB, 2 * D), jnp.float32),
    )


@functools.cache
def _build(B, V, D):
    info = plsc.get_sparse_core_info()
    NC, NS, L = info.num_cores, info.num_subcores, info.num_lanes
    NW = NC * NS
    b_per_w = B // NW
    C = 640
    n_chunks = b_per_w // C
    G = D // L
    M = 2 * D // L
    assert b_per_w % C == 0 and C % L == 0 and n_chunks % 2 == 0

    mesh = plsc.VectorSubcoreMesh(core_axis_name="c", subcore_axis_name="s")

    @functools.partial(
        pl.kernel,
        mesh=mesh,
        out_type=jax.ShapeDtypeStruct((B * G, L), jnp.float32),
        scratch_types=[
            pltpu.VMEM((b_per_w,), jnp.int32),
            pltpu.VMEM((C * G,), jnp.int32),
            pltpu.VMEM((C * G,), jnp.int32),
            pltpu.VMEM((C * G, L), jnp.float32),
            pltpu.VMEM((C * G, L), jnp.float32),
            pltpu.SemaphoreType.DMA,
            pltpu.SemaphoreType.DMA,
            pltpu.SemaphoreType.DMA,
            pltpu.SemaphoreType.DMA,
        ],
        compiler_params=pltpu.CompilerParams(
            use_tc_tiling_on_sc=False, needs_layout_passes=False),
    )
    def gather_scale(idx_hbm, table_hbm, out_hbm,
                     idx_v, i40, i41, rows0, rows1, g0, g1, o0, o1):
        wid = lax.axis_index("s") * NC + lax.axis_index("c")
        base = pl.multiple_of(wid * b_per_w, b_per_w)
        bufs = (rows0, rows1)
        i4s = (i40, i41)
        gsems = (g0, g1)
        osems = (o0, o1)

        pltpu.sync_copy(idx_hbm.at[pl.ds(base, b_per_w)], idx_v)

        iota = lax.iota(jnp.int32, L)
        rks = [lax.shift_right_logical(iota + L * k, 2) for k in range(G)]
        tks = [(iota + L * k) & 3 for k in range(G)]

        def start_gather(ci, b):
            i4 = i4s[b]

            def grp(g, c):
                gb = ci * C + g * L
                for k in range(G):
                    v = plsc.load_gather(idx_v, [gb + rks[k]])
                    row = lax.shift_left(
                        lax.shift_right_logical(v, BK.bit_length() - 1),
                        HB.bit_length() - 1) + (v & (HB - 1))
                    h = lax.shift_right_logical(v, HB.bit_length() - 1) & 1
                    i4[pl.ds(G * L * g + L * k, L)] = (
                        lax.shift_left(row, 3) + lax.shift_left(h, 2) + tks[k])
                return c

            lax.fori_loop(0, C // L, grp, 0)
            pltpu.async_copy(table_hbm.at[i4], bufs[b], gsems[b])

        def gather_wait(b):
            pltpu.make_async_copy(
                table_hbm.at[i4s[b]], bufs[b], gsems[b]).wait()

        def scale(buf):
            def body(q, c):
                r0 = q * L
                for u in range(L):
                    buf[r0 + u, :] = buf[r0 + u, :] * SCALE
                return c

            lax.fori_loop(0, C * G // L, body, 0)

        def start_out(ci, b):
            obase = pl.multiple_of((base + ci * C) * G, C * G)
            pltpu.async_copy(
                bufs[b], out_hbm.at[pl.ds(obase, C * G)], osems[b])

        def out_wait(b):
            pltpu.make_async_copy(
                bufs[b], out_hbm.at[pl.ds(0, C * G)], osems[b]).wait()

        n_pairs = n_chunks // 2
        start_gather(0, 0)
        start_gather(1, 1)

        def pair_body(k, c):
            c0 = 2 * k
            gather_wait(0)
            scale(rows0)
            start_out(c0, 0)
            gather_wait(1)
            scale(rows1)
            out_wait(0)
            start_gather(c0 + 2, 0)
            start_out(c0 + 1, 1)
            out_wait(1)
            start_gather(c0 + 3, 1)
            return c

        lax.fori_loop(0, n_pairs - 1, pair_body, 0)
        cl = n_chunks - 2
        gather_wait(0)
        scale(rows0)
        start_out(cl, 0)
        gather_wait(1)
        scale(rows1)
        start_out(cl + 1, 1)
        out_wait(0)
        out_wait(1)

    return gather_scale


def kernel(x, lut):
    Bt, S = x.shape
    B = Bt * S
    V, D = lut.shape
    xflat = x.reshape(B).astype(jnp.int32)
    tabw = _build_tc_transpose(D, V)(lut.T)
    tab16 = tabw.reshape(tabw.shape[0] * (2 * D // 16), 16)
    out4 = _build(B, V, D)(xflat, tab16)
    return out4.reshape(Bt, S, D)


# speedup vs baseline: 2.1977x; 1.0354x over previous
import functools

import jax
import jax.numpy as jnp
from jax import lax
from jax.experimental import pallas as pl
from jax.experimental.pallas import tpu as pltpu
from jax.experimental.pallas import tpu_sc as plsc

SCALE = 8.0
BK = 32768
HB = BK // 2


@functools.cache
def _build_tc_transpose(D, V):
    grid = (V + BK - 1) // BK

    def body(in_ref, out_ref):
        out_ref[:, 0:D] = jnp.transpose(in_ref[:, 0:HB])
        out_ref[:, D:2 * D] = jnp.transpose(in_ref[:, HB:BK])

    return pl.pallas_call(
        body,
        grid=(grid,),
        in_specs=[pl.BlockSpec((D, BK), lambda g: (0, g))],
        out_specs=pl.BlockSpec((HB, 2 * D), lambda g: (g, 0)),
        out_shape=jax.ShapeDtypeStruct((grid * HB, 2 * D), jnp.float32),
    )


@functools.cache
def _build(B, V, D):
    info = plsc.get_sparse_core_info()
    NC, NS, L = info.num_cores, info.num_subcores, info.num_lanes
    NW = NC * NS
    b_per_w = B // NW
    C = 640
    n_chunks = b_per_w // C
    G = D // L
    M = 2 * D // L
    assert b_per_w % C == 0 and C % L == 0 and n_chunks % 2 == 0

    mesh = plsc.VectorSubcoreMesh(core_axis_name="c", subcore_axis_name="s")

    @functools.partial(
        pl.kernel,
        mesh=mesh,
        out_type=jax.ShapeDtypeStruct((B * G, L), jnp.float32),
        scratch_types=[
            pltpu.VMEM((b_per_w,), jnp.int32),
            pltpu.VMEM((C * G,), jnp.int32),
            pltpu.VMEM((C * G,), jnp.int32),
            pltpu.VMEM((C * G, L), jnp.float32),
            pltpu.VMEM((C * G, L), jnp.float32),
            pltpu.SemaphoreType.DMA,
            pltpu.SemaphoreType.DMA,
            pltpu.SemaphoreType.DMA,
            pltpu.SemaphoreType.DMA,
        ],
        compiler_params=pltpu.CompilerParams(
            use_tc_tiling_on_sc=False, needs_layout_passes=False),
    )
    def gather_scale(idx_hbm, table_hbm, out_hbm,
                     idx_v, i40, i41, rows0, rows1, g0, g1, o0, o1):
        wid = lax.axis_index("s") * NC + lax.axis_index("c")
        base = pl.multiple_of(wid * b_per_w, b_per_w)
        bufs = (rows0, rows1)
        i4s = (i40, i41)
        gsems = (g0, g1)
        osems = (o0, o1)

        pltpu.sync_copy(idx_hbm.at[pl.ds(base, b_per_w)], idx_v)

        iota = lax.iota(jnp.int32, L)
        rks = [lax.shift_right_logical(iota + L * k, 2) for k in range(G)]
        tks = [(iota + L * k) & 3 for k in range(G)]

        def start_gather(ci, b):
            i4 = i4s[b]

            def grp(g, c):
                gb = ci * C + g * L
                for k in range(G):
                    v = plsc.load_gather(idx_v, [gb + rks[k]])
                    row = lax.shift_left(
                        lax.shift_right_logical(v, BK.bit_length() - 1),
                        HB.bit_length() - 1) + (v & (HB - 1))
                    h = lax.shift_right_logical(v, HB.bit_length() - 1) & 1
                    i4[pl.ds(G * L * g + L * k, L)] = (
                        lax.shift_left(row, 3) + lax.shift_left(h, 2) + tks[k])
                return c

            lax.fori_loop(0, C // L, grp, 0)
            pltpu.async_copy(table_hbm.at[i4], bufs[b], gsems[b])

        def gather_wait(b):
            pltpu.make_async_copy(
                table_hbm.at[i4s[b]], bufs[b], gsems[b]).wait()

        def scale(buf):
            def body(q, c):
                r0 = q * L
                for u in range(L):
                    buf[r0 + u, :] = buf[r0 + u, :] * SCALE
                return c

            lax.fori_loop(0, C * G // L, body, 0)

        def start_out(ci, b):
            obase = pl.multiple_of((base + ci * C) * G, C * G)
            pltpu.async_copy(
                bufs[b], out_hbm.at[pl.ds(obase, C * G)], osems[b])

        def out_wait(b):
            pltpu.make_async_copy(
                bufs[b], out_hbm.at[pl.ds(0, C * G)], osems[b]).wait()

        n_pairs = n_chunks // 2
        start_gather(0, 0)
        start_gather(1, 1)

        def pair_body(k, c):
            c0 = 2 * k
            gather_wait(0)
            scale(rows0)
            start_out(c0, 0)
            gather_wait(1)
            scale(rows1)
            out_wait(0)
            start_gather(c0 + 2, 0)
            start_out(c0 + 1, 1)
            out_wait(1)
            start_gather(c0 + 3, 1)
            return c

        lax.fori_loop(0, n_pairs - 1, pair_body, 0)
        cl = n_chunks - 2
        gather_wait(0)
        scale(rows0)
        start_out(cl, 0)
        gather_wait(1)
        scale(rows1)
        start_out(cl + 1, 1)
        out_wait(0)
        out_wait(1)

    return gather_scale


def kernel(x, lut):
    Bt, S = x.shape
    B = Bt * S
    V, D = lut.shape
    xflat = x.reshape(B).astype(jnp.int32)
    tabw = _build_tc_transpose(D, V)(lut.T)
    tab16 = tabw.reshape(tabw.shape[0] * (2 * D // 16), 16)
    out4 = _build(B, V, D)(xflat, tab16)
    return out4.reshape(Bt, S, D)


# speedup vs baseline: 2.7008x; 1.2289x over previous
import functools

import jax
import jax.numpy as jnp
from jax import lax
from jax.experimental import pallas as pl
from jax.experimental.pallas import tpu as pltpu
from jax.experimental.pallas import tpu_sc as plsc

SCALE = 8.0
BK = 32768
HB = BK // 2


@functools.cache
def _build_tc_transpose(D, V):
    grid = (V + BK - 1) // BK

    def body(in_ref, out_ref):
        out_ref[:, 0:D] = jnp.transpose(in_ref[:, 0:HB])
        out_ref[:, D:2 * D] = jnp.transpose(in_ref[:, HB:BK])

    return pl.pallas_call(
        body,
        grid=(grid,),
        in_specs=[pl.BlockSpec((D, BK), lambda g: (0, g))],
        out_specs=pl.BlockSpec((HB, 2 * D), lambda g: (g, 0)),
        out_shape=jax.ShapeDtypeStruct((grid * HB, 2 * D), jnp.float32),
    )


@functools.cache
def _build(B, V, D):
    info = plsc.get_sparse_core_info()
    NC, NS, L = info.num_cores, info.num_subcores, info.num_lanes
    NW = NC * NS
    b_per_w = B // NW
    C = 640
    n_chunks = b_per_w // C
    G = D // L
    M = 2 * D // L
    assert b_per_w % C == 0 and C % L == 0 and n_chunks % 2 == 0

    mesh = plsc.VectorSubcoreMesh(core_axis_name="c", subcore_axis_name="s")

    S = 50
    SP = 56

    @functools.partial(
        pl.kernel,
        mesh=mesh,
        out_type=jax.ShapeDtypeStruct((B // S * SP * M, L), jnp.float32),
        scratch_types=[
            pltpu.VMEM((b_per_w,), jnp.int32),
            pltpu.VMEM((C * G,), jnp.int32),
            pltpu.VMEM((C * G,), jnp.int32),
            pltpu.VMEM((C * G,), jnp.int32),
            pltpu.VMEM((C * G,), jnp.int32),
            pltpu.VMEM((C * G, L), jnp.float32),
            pltpu.VMEM((C * G, L), jnp.float32),
            pltpu.SemaphoreType.DMA,
            pltpu.SemaphoreType.DMA,
            pltpu.SemaphoreType.DMA,
            pltpu.SemaphoreType.DMA,
        ],
        compiler_params=pltpu.CompilerParams(
            use_tc_tiling_on_sc=False, needs_layout_passes=False),
    )
    def gather_scale(idx_hbm, table_hbm, out_hbm,
                     idx_v, i40, i41, o40, o41, rows0, rows1, g0, g1, o0, o1):
        wid = lax.axis_index("s") * NC + lax.axis_index("c")
        base = pl.multiple_of(wid * b_per_w, b_per_w)
        bufs = (rows0, rows1)
        i4s = (i40, i41)
        o4s = (o40, o41)
        gsems = (g0, g1)
        osems = (o0, o1)

        pltpu.sync_copy(idx_hbm.at[pl.ds(base, b_per_w)], idx_v)

        iota = lax.iota(jnp.int32, L)
        rks = [lax.shift_right_logical(iota + L * k, 2) for k in range(G)]
        tks = [(iota + L * k) & 3 for k in range(G)]

        def start_gather(ci, b):
            i4 = i4s[b]
            o4 = o4s[b]

            def grp(g, c):
                gb = ci * C + g * L
                for k in range(G):
                    v = plsc.load_gather(idx_v, [gb + rks[k]])
                    row = lax.shift_left(
                        lax.shift_right_logical(v, BK.bit_length() - 1),
                        HB.bit_length() - 1) + (v & (HB - 1))
                    h = lax.shift_right_logical(v, HB.bit_length() - 1) & 1
                    i4[pl.ds(G * L * g + L * k, L)] = (
                        lax.shift_left(row, 3) + lax.shift_left(h, 2) + tks[k])
                    f = base + gb + rks[k]
                    bb = f // S
                    o4[pl.ds(G * L * g + L * k, L)] = (
                        lax.shift_left(f + (SP - S) * bb, 3) + tks[k])
                return c

            lax.fori_loop(0, C // L, grp, 0)
            pltpu.async_copy(table_hbm.at[i4], bufs[b], gsems[b])

        def gather_wait(b):
            pltpu.make_async_copy(
                table_hbm.at[i4s[b]], bufs[b], gsems[b]).wait()

        def scale(buf):
            def body(q, c):
                r0 = q * L
                for u in range(L):
                    buf[r0 + u, :] = buf[r0 + u, :] * SCALE
                return c

            lax.fori_loop(0, C * G // L, body, 0)

        def start_out(ci, b):
            pltpu.async_copy(bufs[b], out_hbm.at[o4s[b]], osems[b])

        def out_wait(b):
            pltpu.make_async_copy(
                bufs[b], out_hbm.at[o4s[b]], osems[b]).wait()

        n_pairs = n_chunks // 2
        start_gather(0, 0)
        start_gather(1, 1)

        def pair_body(k, c):
            c0 = 2 * k
            gather_wait(0)
            scale(rows0)
            start_out(c0, 0)
            gather_wait(1)
            scale(rows1)
            out_wait(0)
            start_gather(c0 + 2, 0)
            start_out(c0 + 1, 1)
            out_wait(1)
            start_gather(c0 + 3, 1)
            return c

        lax.fori_loop(0, n_pairs - 1, pair_body, 0)
        cl = n_chunks - 2
        gather_wait(0)
        scale(rows0)
        start_out(cl, 0)
        gather_wait(1)
        scale(rows1)
        start_out(cl + 1, 1)
        out_wait(0)
        out_wait(1)

    return gather_scale


def kernel(x, lut):
    Bt, S = x.shape
    B = Bt * S
    V, D = lut.shape
    xflat = x.reshape(B).astype(jnp.int32)
    tabw = _build_tc_transpose(D, V)(lut.T)
    tab16 = tabw.reshape(tabw.shape[0] * (2 * D // 16), 16)
    outp = _build(B, V, D)(xflat, tab16)
    return outp.reshape(Bt, 56, 2 * D)[:, :S, :D]
